# baseline (speedup 1.0000x reference)
import jax
import jax.numpy as jnp
from jax.experimental import pallas as pl

SEQ_BLOCK = 256


def _posemb_add_body(x_ref, emb_ref, o_ref):
    i = pl.program_id(0)
    win = emb_ref[pl.ds((i * SEQ_BLOCK + 1) * 8, SEQ_BLOCK * 8), :]
    emb = win.reshape(SEQ_BLOCK, 8, 128)
    o_ref[...] = x_ref[...] + emb[:, None, :, :]


def kernel(inputs, pos_emb):
    S, B, D = inputs.shape
    T = pos_emb.shape[0]
    x4 = inputs.reshape(S, B, 8, D // 8)
    e2 = pos_emb.reshape(T * 8, D // 8)
    grid = (S // SEQ_BLOCK,)
    out = pl.pallas_call(
        _posemb_add_body,
        grid=grid,
        in_specs=[
            pl.BlockSpec((SEQ_BLOCK, B, 8, D // 8), lambda i: (i, 0, 0, 0)),
            pl.BlockSpec((T * 8, D // 8), lambda i: (0, 0)),
        ],
        out_specs=pl.BlockSpec((SEQ_BLOCK, B, 8, D // 8), lambda i: (i, 0, 0, 0)),
        out_shape=jax.ShapeDtypeStruct((S, B, 8, D // 8), inputs.dtype),
    )(x4, e2)
    return out.reshape(S, B, D)


# speedup vs baseline: 1.0004x; 1.0004x over previous
import jax
import jax.numpy as jnp
from jax.experimental import pallas as pl

SEQ_BLOCK = 256


def _posemb_add_body(x_ref, ea_ref, eb_ref, o_ref):
    win = jnp.concatenate(
        [ea_ref[pl.ds(8, 8 * SEQ_BLOCK - 8), :], eb_ref[...]], axis=0
    )
    emb = win.reshape(SEQ_BLOCK, 8, 128)
    o_ref[...] = x_ref[...] + emb[:, None, :, :]


def kernel(inputs, pos_emb):
    S, B, D = inputs.shape
    T = pos_emb.shape[0]
    x4 = inputs.reshape(S, B, 8, D // 8)
    e2 = pos_emb.reshape(T * 8, D // 8)
    grid = (S // SEQ_BLOCK,)
    out = pl.pallas_call(
        _posemb_add_body,
        grid=grid,
        in_specs=[
            pl.BlockSpec((SEQ_BLOCK, B, 8, D // 8), lambda i: (i, 0, 0, 0)),
            pl.BlockSpec((8 * SEQ_BLOCK, D // 8), lambda i: (i, 0)),
            pl.BlockSpec((8, D // 8), lambda i: ((i + 1) * SEQ_BLOCK, 0)),
        ],
        out_specs=pl.BlockSpec((SEQ_BLOCK, B, 8, D // 8), lambda i: (i, 0, 0, 0)),
        out_shape=jax.ShapeDtypeStruct((S, B, 8, D // 8), inputs.dtype),
    )(x4, e2, e2)
    return out.reshape(S, B, D)
